# Initial kernel scaffold; baseline (speedup 1.0000x reference)
import jax
import jax.numpy as jnp
from jax.experimental import pallas as pl


def kernel(x, edge_index, W1, b1, W2, b2, Wih0, Whh0, bih0, bhh0, Wih1, Whh1, bih1, bhh1, Wc, bc):
    raise NotImplementedError("write your pallas kernel here")


# speedup vs baseline: 12.1875x; 12.1875x over previous
import functools

import jax
import jax.numpy as jnp
from jax import lax
from jax.experimental import pallas as pl
from jax.experimental.pallas import tpu as pltpu
from jax.experimental.pallas import tpu_sc as plsc

T, N, D, H, OUT = 8, 10000, 128, 128, 2
E = 320000
NC, NS = 2, 16
NACC = 10240
RPT = NACC // NS
EROWS = 2560
EPAD = EROWS * 128
KB = 16
TPC = T // NC
F32 = jnp.float32
HIGH = lax.Precision.HIGHEST

_mesh = plsc.VectorSubcoreMesh(core_axis_name="c", subcore_axis_name="s")


def _sig(v):
    return 1.0 / (1.0 + jnp.exp(-v))


@functools.partial(
    pl.kernel, mesh=_mesh,
    out_type=jax.ShapeDtypeStruct((NC, NACC, 128), F32),
    scratch_types=[
        pltpu.VMEM((KB, 128), jnp.int32),
        pltpu.VMEM((128, 128), F32),
        pltpu.VMEM((128, 128), F32),
        pltpu.VMEM_SHARED((NACC, 128), F32),
    ])
def _degree(dstr, ones_c, zeros_c, out, dstv, onesv, z16v, dacc):
    c = lax.axis_index("c")
    s = lax.axis_index("s")
    w = s * NC + c
    pltpu.sync_copy(ones_c, onesv)
    pltpu.sync_copy(zeros_c, z16v)
    for z in range(RPT // 128):
        pltpu.sync_copy(z16v, dacc.at[pl.ds(s * RPT + z * 128, 128)])
    plsc.subcore_barrier()
    rpt_e = EROWS // (NC * NS)

    def body(b, carry):
        row0 = w * rpt_e + b * KB
        pltpu.sync_copy(dstr.at[pl.ds(row0, KB)], dstv)
        for j in range(KB):
            pltpu.sync_copy(onesv, dacc.at[dstv.at[j]], add=True)
        return carry

    lax.fori_loop(0, rpt_e // KB, body, 0)
    plsc.subcore_barrier()
    pltpu.sync_copy(dacc.at[pl.ds(s * RPT, RPT)], out.at[c, pl.ds(s * RPT, RPT)])


@functools.partial(
    pl.kernel, mesh=_mesh,
    out_type=jax.ShapeDtypeStruct((T, NACC, H), F32),
    scratch_types=[
        pltpu.VMEM((KB, 128), jnp.int32),
        pltpu.VMEM((KB, 128), jnp.int32),
        pltpu.VMEM((128, H), F32),
        pltpu.VMEM((128, H), F32),
        pltpu.VMEM_SHARED((NACC, H), F32),
        pltpu.SemaphoreType.DMA,
    ])
def _propagate(S, srcr, dstr, zeros_c, out, srcv, dstv, zbuf, rows, acc, sem):
    c = lax.axis_index("c")
    s = lax.axis_index("s")
    pltpu.sync_copy(zeros_c, zbuf)
    rpt_e = EROWS // NS
    for tt in range(TPC):
        tg = c * TPC + tt
        for z in range(RPT // 128):
            pltpu.sync_copy(zbuf, acc.at[pl.ds(s * RPT + z * 128, 128)])
        plsc.subcore_barrier()

        def body(b, carry):
            row0 = s * rpt_e + b * KB
            pltpu.sync_copy(srcr.at[pl.ds(row0, KB)], srcv)
            pltpu.sync_copy(dstr.at[pl.ds(row0, KB)], dstv)
            for j in range(KB):
                pltpu.async_copy(S.at[tg].at[srcv.at[j]], rows, sem).wait()
                pltpu.sync_copy(rows, acc.at[dstv.at[j]], add=True)
            return carry

        lax.fori_loop(0, rpt_e // KB, body, 0)
        plsc.subcore_barrier()
        pltpu.sync_copy(acc.at[pl.ds(s * RPT, RPT)],
                        out.at[tg, pl.ds(s * RPT, RPT)])
        plsc.subcore_barrier()


BN1 = 1000
BN3 = 1000


def _dinv_body(dg_ref, o_ref):
    v = dg_ref[...]
    dsum = v[0, :N, 0] + v[1, :N, 0] + 1.0
    o_ref[...] = lax.rsqrt(jnp.maximum(dsum, 1e-12))[:, None]


def _dinv(degw):
    return pl.pallas_call(
        _dinv_body,
        grid=(1,),
        in_specs=[pl.BlockSpec((NC, NACC, 128), lambda i: (0, 0, 0))],
        out_specs=pl.BlockSpec((N, 1), lambda i: (0, 0)),
        out_shape=jax.ShapeDtypeStruct((N, 1), F32),
    )(degw)


def _m1_body(x_ref, w_ref, dinv_ref, o_ref):
    o_ref[0] = jnp.dot(x_ref[0], w_ref[...], preferred_element_type=F32,
                       precision=HIGH) * dinv_ref[...]


def _m1(x, W1, dinv):
    return pl.pallas_call(
        _m1_body,
        grid=(T, N // BN1),
        in_specs=[pl.BlockSpec((1, BN1, D), lambda t, i: (t, i, 0)),
                  pl.BlockSpec((D, H), lambda t, i: (0, 0)),
                  pl.BlockSpec((BN1, 1), lambda t, i: (i, 0))],
        out_specs=pl.BlockSpec((1, BN1, H), lambda t, i: (t, i, 0)),
        out_shape=jax.ShapeDtypeStruct((T, N, H), F32),
    )(x, W1, dinv)


def _m2_body(agg_ref, s1_ref, dinv_ref, b_ref, w_ref, o_ref):
    dinv = dinv_ref[...]
    h = jnp.maximum((agg_ref[0] + s1_ref[0]) * dinv + b_ref[...], 0.0)
    o_ref[0] = jnp.dot(h, w_ref[...], preferred_element_type=F32,
                       precision=HIGH) * dinv


def _m2(agg1, S1, dinv, b1, W2):
    return pl.pallas_call(
        _m2_body,
        grid=(T, N // BN1),
        in_specs=[pl.BlockSpec((1, BN1, H), lambda t, i: (t, i, 0)),
                  pl.BlockSpec((1, BN1, H), lambda t, i: (t, i, 0)),
                  pl.BlockSpec((BN1, 1), lambda t, i: (i, 0)),
                  pl.BlockSpec((1, H), lambda t, i: (0, 0)),
                  pl.BlockSpec((H, H), lambda t, i: (0, 0))],
        out_specs=pl.BlockSpec((1, BN1, H), lambda t, i: (t, i, 0)),
        out_shape=jax.ShapeDtypeStruct((T, N, H), F32),
    )(agg1, S1, dinv, b1, W2)


def _lstm_body(agg_ref, s2_ref, dinv_ref, b2_ref,
               wih0_ref, whh0_ref, bih0_ref, bhh0_ref,
               wih1_ref, whh1_ref, bih1_ref, bhh1_ref,
               wcp_ref, bcp_ref, emb_ref, outp_ref):
    dinv = dinv_ref[...]
    b2v = b2_ref[...]
    wih0 = wih0_ref[...]
    whh0 = whh0_ref[...]
    b0 = bih0_ref[...] + bhh0_ref[...]
    wih1 = wih1_ref[...]
    whh1 = whh1_ref[...]
    b1g = bih1_ref[...] + bhh1_ref[...]
    h0 = jnp.zeros((BN3, H), F32)
    c0 = jnp.zeros((BN3, H), F32)
    h1 = jnp.zeros((BN3, H), F32)
    c1 = jnp.zeros((BN3, H), F32)
    for t in range(T):
        xt = jnp.maximum((agg_ref[t] + s2_ref[t]) * dinv + b2v, 0.0)
        g = (jnp.dot(xt, wih0, preferred_element_type=F32, precision=HIGH)
             + jnp.dot(h0, whh0, preferred_element_type=F32, precision=HIGH)
             + b0)
        c0 = _sig(g[:, H:2 * H]) * c0 + _sig(g[:, :H]) * jnp.tanh(g[:, 2 * H:3 * H])
        h0 = _sig(g[:, 3 * H:]) * jnp.tanh(c0)
        g = (jnp.dot(h0, wih1, preferred_element_type=F32, precision=HIGH)
             + jnp.dot(h1, whh1, preferred_element_type=F32, precision=HIGH)
             + b1g)
        c1 = _sig(g[:, H:2 * H]) * c1 + _sig(g[:, :H]) * jnp.tanh(g[:, 2 * H:3 * H])
        h1 = _sig(g[:, 3 * H:]) * jnp.tanh(c1)
    emb_ref[...] = h1
    outp_ref[...] = jnp.dot(h1, wcp_ref[...], preferred_element_type=F32,
                            precision=HIGH) + bcp_ref[...]


def _lstm(agg2, S2, dinv, b2, Wih0, Whh0, bih0, bhh0,
          Wih1, Whh1, bih1, bhh1, Wcp, bcp):
    cfull = lambda shape: pl.BlockSpec(shape, lambda i: tuple(0 for _ in shape))
    return pl.pallas_call(
        _lstm_body,
        grid=(N // BN3,),
        in_specs=[pl.BlockSpec((T, BN3, H), lambda i: (0, i, 0)),
                  pl.BlockSpec((T, BN3, H), lambda i: (0, i, 0)),
                  pl.BlockSpec((BN3, 1), lambda i: (i, 0)),
                  cfull((1, H)),
                  cfull((H, 4 * H)), cfull((H, 4 * H)),
                  cfull((1, 4 * H)), cfull((1, 4 * H)),
                  cfull((H, 4 * H)), cfull((H, 4 * H)),
                  cfull((1, 4 * H)), cfull((1, 4 * H)),
                  cfull((H, 128)), cfull((1, 128))],
        out_specs=[pl.BlockSpec((BN3, H), lambda i: (i, 0)),
                   pl.BlockSpec((BN3, 128), lambda i: (i, 0))],
        out_shape=[jax.ShapeDtypeStruct((N, H), F32),
                   jax.ShapeDtypeStruct((N, 128), F32)],
    )(agg2, S2, dinv, b2, Wih0, Whh0, bih0, bhh0,
      Wih1, Whh1, bih1, bhh1, Wcp, bcp)


def kernel(x, edge_index, W1, b1, W2, b2, Wih0, Whh0, bih0, bhh0,
           Wih1, Whh1, bih1, bhh1, Wc, bc):
    src = edge_index[0]
    dst = edge_index[1]
    pad = EPAD - E
    srcr = jnp.concatenate([src, jnp.zeros((pad,), jnp.int32)]).reshape(EROWS, 128)
    dstr = jnp.concatenate([dst, jnp.full((pad,), N, jnp.int32)]).reshape(EROWS, 128)
    zeros_c = jnp.zeros((128, H), F32)
    ones_c = jnp.ones((128, 128), F32)

    degw = _degree(dstr, ones_c, zeros_c)
    dinv = _dinv(degw)
    S1 = _m1(x, W1, dinv)
    agg1 = _propagate(S1, srcr, dstr, zeros_c)
    S2 = _m2(agg1, S1, dinv, b1.reshape(1, H), W2)
    agg2 = _propagate(S2, srcr, dstr, zeros_c)

    Wcp = jnp.zeros((H, 128), F32).at[:, :OUT].set(Wc)
    bcp = jnp.zeros((1, 128), F32).at[0, :OUT].set(bc)
    emb, outp = _lstm(agg2, S2, dinv, b2.reshape(1, H),
                      Wih0, Whh0, bih0.reshape(1, 4 * H), bhh0.reshape(1, 4 * H),
                      Wih1, Whh1, bih1.reshape(1, 4 * H), bhh1.reshape(1, 4 * H),
                      Wcp, bcp)
    return (outp[:, :OUT], emb)


# speedup vs baseline: 14.3647x; 1.1786x over previous
import functools

import jax
import jax.numpy as jnp
from jax import lax
from jax.experimental import pallas as pl
from jax.experimental.pallas import tpu as pltpu
from jax.experimental.pallas import tpu_sc as plsc

T, N, D, H, OUT = 8, 10000, 128, 128, 2
E = 320000
NC, NS = 2, 16
NACC = 10240
RPT = NACC // NS
EROWS = 2560
EPAD = EROWS * 128
KB = 16
TPC = T // NC
F32 = jnp.float32
HIGH = lax.Precision.HIGHEST

_mesh = plsc.VectorSubcoreMesh(core_axis_name="c", subcore_axis_name="s")


def _sig(v):
    return 1.0 / (1.0 + jnp.exp(-v))


@functools.partial(
    pl.kernel, mesh=_mesh,
    out_type=jax.ShapeDtypeStruct((NC, NACC, 128), F32),
    scratch_types=[
        pltpu.VMEM((KB, 128), jnp.int32),
        pltpu.VMEM((128, 128), F32),
        pltpu.VMEM((128, 128), F32),
        pltpu.VMEM_SHARED((NACC, 128), F32),
    ])
def _degree(dstr, ones_c, zeros_c, out, dstv, onesv, z16v, dacc):
    c = lax.axis_index("c")
    s = lax.axis_index("s")
    w = s * NC + c
    pltpu.sync_copy(ones_c, onesv)
    pltpu.sync_copy(zeros_c, z16v)
    for z in range(RPT // 128):
        pltpu.sync_copy(z16v, dacc.at[pl.ds(s * RPT + z * 128, 128)])
    plsc.subcore_barrier()
    rpt_e = EROWS // (NC * NS)

    def body(b, carry):
        row0 = w * rpt_e + b * KB
        pltpu.sync_copy(dstr.at[pl.ds(row0, KB)], dstv)
        for j in range(KB):
            pltpu.sync_copy(onesv, dacc.at[dstv.at[j]], add=True)
        return carry

    lax.fori_loop(0, rpt_e // KB, body, 0)
    plsc.subcore_barrier()
    pltpu.sync_copy(dacc.at[pl.ds(s * RPT, RPT)], out.at[c, pl.ds(s * RPT, RPT)])


@functools.partial(
    pl.kernel, mesh=_mesh,
    out_type=jax.ShapeDtypeStruct((T, NACC, H), F32),
    scratch_types=[
        pltpu.VMEM((KB, 128), jnp.int32),
        pltpu.VMEM((KB, 128), jnp.int32),
        pltpu.VMEM((128, H), F32),
        pltpu.VMEM((128, H), F32),
        pltpu.VMEM_SHARED((NACC, H), F32),
        pltpu.SemaphoreType.DMA,
        pltpu.SemaphoreType.DMA,
    ])
def _propagate(S, srcr, dstr, zeros_c, out, srcv, dstv, rows0, rows1,
               acc, sem0, sem1):
    c = lax.axis_index("c")
    s = lax.axis_index("s")
    rpt_e = EROWS // NS
    rows = (rows0, rows1)
    sems = (sem0, sem1)
    for tt in range(TPC):
        tg = c * TPC + tt
        pltpu.sync_copy(zeros_c, rows0)
        for z in range(RPT // 128):
            pltpu.sync_copy(rows0, acc.at[pl.ds(s * RPT + z * 128, 128)])
        plsc.subcore_barrier()

        def body(b, carry):
            row0 = s * rpt_e + b * KB
            pltpu.sync_copy(srcr.at[pl.ds(row0, KB)], srcv)
            pltpu.sync_copy(dstr.at[pl.ds(row0, KB)], dstv)
            cps = [pltpu.async_copy(S.at[tg].at[srcv.at[0]], rows0, sem0), None]
            for j in range(KB):
                if j + 1 < KB:
                    k = (j + 1) % 2
                    cps[k] = pltpu.async_copy(S.at[tg].at[srcv.at[j + 1]],
                                              rows[k], sems[k])
                cps[j % 2].wait()
                pltpu.sync_copy(rows[j % 2], acc.at[dstv.at[j]], add=True)
            return carry

        lax.fori_loop(0, rpt_e // KB, body, 0)
        plsc.subcore_barrier()
        pltpu.sync_copy(acc.at[pl.ds(s * RPT, RPT)],
                        out.at[tg, pl.ds(s * RPT, RPT)])
        plsc.subcore_barrier()


BN1 = 1000
BN3 = 1000


def _dinv_body(dg_ref, o_ref):
    v = dg_ref[...]
    dsum = v[0, :N, 0] + v[1, :N, 0] + 1.0
    o_ref[...] = lax.rsqrt(jnp.maximum(dsum, 1e-12))[:, None]


def _dinv(degw):
    return pl.pallas_call(
        _dinv_body,
        grid=(1,),
        in_specs=[pl.BlockSpec((NC, NACC, 128), lambda i: (0, 0, 0))],
        out_specs=pl.BlockSpec((N, 1), lambda i: (0, 0)),
        out_shape=jax.ShapeDtypeStruct((N, 1), F32),
    )(degw)


def _m1_body(x_ref, w_ref, dinv_ref, o_ref):
    o_ref[0] = jnp.dot(x_ref[0], w_ref[...], preferred_element_type=F32,
                       precision=HIGH) * dinv_ref[...]


def _m1(x, W1, dinv):
    return pl.pallas_call(
        _m1_body,
        grid=(T, N // BN1),
        in_specs=[pl.BlockSpec((1, BN1, D), lambda t, i: (t, i, 0)),
                  pl.BlockSpec((D, H), lambda t, i: (0, 0)),
                  pl.BlockSpec((BN1, 1), lambda t, i: (i, 0))],
        out_specs=pl.BlockSpec((1, BN1, H), lambda t, i: (t, i, 0)),
        out_shape=jax.ShapeDtypeStruct((T, N, H), F32),
    )(x, W1, dinv)


def _m2_body(agg_ref, s1_ref, dinv_ref, b_ref, w_ref, o_ref):
    dinv = dinv_ref[...]
    h = jnp.maximum((agg_ref[0] + s1_ref[0]) * dinv + b_ref[...], 0.0)
    o_ref[0] = jnp.dot(h, w_ref[...], preferred_element_type=F32,
                       precision=HIGH) * dinv


def _m2(agg1, S1, dinv, b1, W2):
    return pl.pallas_call(
        _m2_body,
        grid=(T, N // BN1),
        in_specs=[pl.BlockSpec((1, BN1, H), lambda t, i: (t, i, 0)),
                  pl.BlockSpec((1, BN1, H), lambda t, i: (t, i, 0)),
                  pl.BlockSpec((BN1, 1), lambda t, i: (i, 0)),
                  pl.BlockSpec((1, H), lambda t, i: (0, 0)),
                  pl.BlockSpec((H, H), lambda t, i: (0, 0))],
        out_specs=pl.BlockSpec((1, BN1, H), lambda t, i: (t, i, 0)),
        out_shape=jax.ShapeDtypeStruct((T, N, H), F32),
    )(agg1, S1, dinv, b1, W2)


def _lstm_body(agg_ref, s2_ref, dinv_ref, b2_ref,
               wih0_ref, whh0_ref, bih0_ref, bhh0_ref,
               wih1_ref, whh1_ref, bih1_ref, bhh1_ref,
               wcp_ref, bcp_ref, emb_ref, outp_ref):
    dinv = dinv_ref[...]
    b2v = b2_ref[...]
    wih0 = wih0_ref[...]
    whh0 = whh0_ref[...]
    b0 = bih0_ref[...] + bhh0_ref[...]
    wih1 = wih1_ref[...]
    whh1 = whh1_ref[...]
    b1g = bih1_ref[...] + bhh1_ref[...]
    h0 = jnp.zeros((BN3, H), F32)
    c0 = jnp.zeros((BN3, H), F32)
    h1 = jnp.zeros((BN3, H), F32)
    c1 = jnp.zeros((BN3, H), F32)
    for t in range(T):
        xt = jnp.maximum((agg_ref[t] + s2_ref[t]) * dinv + b2v, 0.0)
        g = (jnp.dot(xt, wih0, preferred_element_type=F32, precision=HIGH)
             + jnp.dot(h0, whh0, preferred_element_type=F32, precision=HIGH)
             + b0)
        c0 = _sig(g[:, H:2 * H]) * c0 + _sig(g[:, :H]) * jnp.tanh(g[:, 2 * H:3 * H])
        h0 = _sig(g[:, 3 * H:]) * jnp.tanh(c0)
        g = (jnp.dot(h0, wih1, preferred_element_type=F32, precision=HIGH)
             + jnp.dot(h1, whh1, preferred_element_type=F32, precision=HIGH)
             + b1g)
        c1 = _sig(g[:, H:2 * H]) * c1 + _sig(g[:, :H]) * jnp.tanh(g[:, 2 * H:3 * H])
        h1 = _sig(g[:, 3 * H:]) * jnp.tanh(c1)
    emb_ref[...] = h1
    outp_ref[...] = jnp.dot(h1, wcp_ref[...], preferred_element_type=F32,
                            precision=HIGH) + bcp_ref[...]


def _lstm(agg2, S2, dinv, b2, Wih0, Whh0, bih0, bhh0,
          Wih1, Whh1, bih1, bhh1, Wcp, bcp):
    cfull = lambda shape: pl.BlockSpec(shape, lambda i: tuple(0 for _ in shape))
    return pl.pallas_call(
        _lstm_body,
        grid=(N // BN3,),
        in_specs=[pl.BlockSpec((T, BN3, H), lambda i: (0, i, 0)),
                  pl.BlockSpec((T, BN3, H), lambda i: (0, i, 0)),
                  pl.BlockSpec((BN3, 1), lambda i: (i, 0)),
                  cfull((1, H)),
                  cfull((H, 4 * H)), cfull((H, 4 * H)),
                  cfull((1, 4 * H)), cfull((1, 4 * H)),
                  cfull((H, 4 * H)), cfull((H, 4 * H)),
                  cfull((1, 4 * H)), cfull((1, 4 * H)),
                  cfull((H, 128)), cfull((1, 128))],
        out_specs=[pl.BlockSpec((BN3, H), lambda i: (i, 0)),
                   pl.BlockSpec((BN3, 128), lambda i: (i, 0))],
        out_shape=[jax.ShapeDtypeStruct((N, H), F32),
                   jax.ShapeDtypeStruct((N, 128), F32)],
    )(agg2, S2, dinv, b2, Wih0, Whh0, bih0, bhh0,
      Wih1, Whh1, bih1, bhh1, Wcp, bcp)


def kernel(x, edge_index, W1, b1, W2, b2, Wih0, Whh0, bih0, bhh0,
           Wih1, Whh1, bih1, bhh1, Wc, bc):
    src = edge_index[0]
    dst = edge_index[1]
    pad = EPAD - E
    srcr = jnp.concatenate([src, jnp.zeros((pad,), jnp.int32)]).reshape(EROWS, 128)
    dstr = jnp.concatenate([dst, jnp.full((pad,), N, jnp.int32)]).reshape(EROWS, 128)
    zeros_c = jnp.zeros((128, H), F32)
    ones_c = jnp.ones((128, 128), F32)

    degw = _degree(dstr, ones_c, zeros_c)
    dinv = _dinv(degw)
    S1 = _m1(x, W1, dinv)
    agg1 = _propagate(S1, srcr, dstr, zeros_c)
    S2 = _m2(agg1, S1, dinv, b1.reshape(1, H), W2)
    agg2 = _propagate(S2, srcr, dstr, zeros_c)

    Wcp = jnp.zeros((H, 128), F32).at[:, :OUT].set(Wc)
    bcp = jnp.zeros((1, 128), F32).at[0, :OUT].set(bc)
    emb, outp = _lstm(agg2, S2, dinv, b2.reshape(1, H),
                      Wih0, Whh0, bih0.reshape(1, 4 * H), bhh0.reshape(1, 4 * H),
                      Wih1, Whh1, bih1.reshape(1, 4 * H), bhh1.reshape(1, 4 * H),
                      Wcp, bcp)
    return (outp[:, :OUT], emb)
